# baseline (speedup 1.0000x reference)
import numpy as np

import jax
import jax.numpy as jnp
from jax.experimental import pallas as pl

TA = 64
TB = 512
K = 512


def _conv_feats_kernel(x_ref, w1t_ref, b1t_ref, w2_ref, b2_ref, out_ref):
    ta = TA
    x = x_ref[...].reshape(7 * ta, 112)
    z1 = jnp.zeros((ta, 112), jnp.float32)
    xe = jnp.concatenate([z1, x, z1], axis=0)
    prev = xe[0:7 * ta, 84:112]
    cur = xe[ta:8 * ta]
    nxt = xe[2 * ta:9 * ta, 0:28]
    p1 = jnp.concatenate([prev, cur, nxt], axis=1)
    h1 = jnp.dot(p1, w1t_ref[...], preferred_element_type=jnp.float32)
    h1 = jnp.maximum(h1 + b1t_ref[...], 0.0)
    hq = jnp.concatenate(
        [jnp.maximum(h1[:, 0:896], h1[:, 896:1792]),
         jnp.maximum(h1[:, 1792:2688], h1[:, 2688:3584])], axis=1)
    p2pre = jnp.concatenate(
        [jnp.concatenate(
            [jnp.maximum(hq[:, s2 * 896 + (2 * u) * 32:s2 * 896 + (2 * u) * 32 + 32],
                         hq[:, s2 * 896 + (2 * u + 1) * 32:s2 * 896 + (2 * u + 1) * 32 + 32])
             for s2 in range(2)], axis=1)
         for u in range(14)], axis=0)
    bw = 7 * ta
    el = 14 * bw
    zb = jnp.zeros((8 * ta, 64), jnp.float32)
    ppad = jnp.concatenate([zb, p2pre, zb], axis=0)
    yq = jax.lax.broadcasted_iota(jnp.int32, (el, 1), 0)
    y2p = (yq // ta) % 7
    s0 = 8 * ta
    slices = []
    for dx in (-1, 0, 1):
        b0 = s0 + dx * bw
        a = jnp.where(y2p == 0, 0.0, ppad[b0 - ta:b0 - ta + el, 32:64])
        bm = ppad[b0:b0 + el]
        c = jnp.where(y2p == 6, 0.0, ppad[b0 + ta:b0 + ta + el, 0:32])
        slices += [a, bm, c]
    p2 = jnp.concatenate(slices, axis=1)
    h2 = jnp.dot(p2, w2_ref[...], preferred_element_type=jnp.float32)
    h2 = jnp.maximum(h2 + b2_ref[...], 0.0)
    hyp = jnp.maximum(h2[:, :64], h2[:, 64:])
    h2p = jnp.concatenate(
        [jnp.maximum(hyp[(2 * u) * bw:(2 * u + 1) * bw],
                     hyp[(2 * u + 1) * bw:(2 * u + 2) * bw])
         for u in range(7)] + [jnp.zeros((bw, 64), jnp.float32)],
        axis=1)
    out_ref[...] = h2p.reshape(7, ta, 512)


def _fc_vq_kernel(f_ref, fcw_ref, fcb_ref, cct_ref, fc_out_ref, prob_ref, w_ref):
    t = f_ref.shape[1]
    f = fcb_ref[...]
    for y2p in range(7):
        f = f + jnp.dot(f_ref[y2p], fcw_ref[y2p],
                        preferred_element_type=jnp.float32)
    fc_out_ref[...] = f
    cct = cct_ref[...]
    a2 = jnp.sum(f * f, axis=1, keepdims=True)
    b2 = jnp.sum(cct * cct, axis=0, keepdims=True)
    fg = jnp.dot(f, cct, preferred_element_type=jnp.float32)
    d2 = jnp.maximum(a2 + b2 - 2.0 * fg, 0.0)
    dist = jnp.sqrt(d2 + 1e-12)
    dmin = jnp.min(dist, axis=1, keepdims=True)
    iota = jax.lax.broadcasted_iota(jnp.int32, (t, K), 1)
    label = jnp.min(jnp.where(dist == dmin, iota, K), axis=1, keepdims=True)
    w_ref[...] = (iota == label).astype(jnp.float32)
    z = -dist
    z = z - jnp.max(z, axis=1, keepdims=True)
    e = jnp.exp(z)
    prob_ref[...] = e / jnp.sum(e, axis=1, keepdims=True)


def kernel(x, conv1_w, conv1_b, conv2_w, conv2_b, fc_w, fc_b, cluster_centers):
    b = x.shape[0]
    x5 = x.reshape(b, 7, 4, 28).transpose(1, 0, 2, 3).reshape(7, b, 112)
    w1t = jnp.zeros((6, 28, 2, 2, 28, 32), jnp.float32)
    xs = np.arange(28)
    for yy in range(6):
        for s2 in range(2):
            for s in range(2):
                ky = yy - 2 * s2 - s
                if not 0 <= ky < 3:
                    continue
                for kx in range(3):
                    xv = xs - (kx - 1)
                    valid = (xv >= 0) & (xv < 28)
                    w1t = w1t.at[yy, xs[valid], s2, s, xv[valid], :].set(
                        conv1_w[:, 0, ky, kx])
    w1t = w1t.reshape(168, 3584)
    b1t = jnp.tile(conv1_b, 112).reshape(1, 3584)
    w2r = jnp.zeros((3, 4, 32, 2, 64), jnp.float32)
    for kx in range(3):
        for yy in range(4):
            for s2o in range(2):
                ky = yy - s2o
                if 0 <= ky < 3:
                    w2r = w2r.at[kx, yy, :, s2o, :].set(conv2_w[:, :, ky, kx].T)
    w2r = w2r.reshape(384, 128)
    b2 = jnp.tile(conv2_b, 2).reshape(1, 128)
    fcw = jnp.zeros((7, 512, K), jnp.float32)
    fcw = fcw.at[:, :448].set(
        fc_w.reshape(K, 64, 7, 7).transpose(2, 3, 1, 0).reshape(7, 448, K))
    fcb = fc_b.reshape(1, K)
    cct = cluster_centers.T

    feats_alt = pl.pallas_call(
        _conv_feats_kernel,
        grid=(b // TA,),
        in_specs=[
            pl.BlockSpec((7, TA, 112), lambda i: (0, i, 0)),
            pl.BlockSpec((168, 3584), lambda i: (0, 0)),
            pl.BlockSpec((1, 3584), lambda i: (0, 0)),
            pl.BlockSpec((384, 128), lambda i: (0, 0)),
            pl.BlockSpec((1, 128), lambda i: (0, 0)),
        ],
        out_specs=pl.BlockSpec((7, TA, 512), lambda i: (0, i, 0)),
        out_shape=jax.ShapeDtypeStruct((7, b, 512), jnp.float32),
    )(x5, w1t, b1t, w2r, b2)

    fc_out, prob, w = pl.pallas_call(
        _fc_vq_kernel,
        grid=(b // TB,),
        in_specs=[
            pl.BlockSpec((7, TB, 512), lambda i: (0, i, 0)),
            pl.BlockSpec((7, 512, K), lambda i: (0, 0, 0)),
            pl.BlockSpec((1, K), lambda i: (0, 0)),
            pl.BlockSpec((K, K), lambda i: (0, 0)),
        ],
        out_specs=[
            pl.BlockSpec((TB, K), lambda i: (i, 0)),
            pl.BlockSpec((TB, K), lambda i: (i, 0)),
            pl.BlockSpec((TB, K), lambda i: (i, 0)),
        ],
        out_shape=[
            jax.ShapeDtypeStruct((b, K), jnp.float32),
            jax.ShapeDtypeStruct((b, K), jnp.float32),
            jax.ShapeDtypeStruct((b, K), jnp.float32),
        ],
    )(feats_alt, fcw, fcb, cct)
    return (fc_out, prob, w)


# speedup vs baseline: 1.4142x; 1.4142x over previous
import numpy as np

import jax
import jax.numpy as jnp
from jax.experimental import pallas as pl

TA = 64
TB = 512
K = 512


def _conv_feats_kernel(x_ref, w1t_ref, b1t_ref, w2_ref, b2_ref, out_ref):
    ta = TA
    x = x_ref[...].reshape(7 * ta, 112)
    z1 = jnp.zeros((ta, 112), jnp.float32)
    xe = jnp.concatenate([z1, x, z1], axis=0)
    prev = xe[0:7 * ta, 84:112]
    cur = xe[ta:8 * ta]
    nxt = xe[2 * ta:9 * ta, 0:28]
    p1 = jnp.concatenate([prev, cur, nxt], axis=1)
    h1 = jnp.dot(p1, w1t_ref[...], preferred_element_type=jnp.float32)
    h1 = jnp.maximum(h1 + b1t_ref[...], 0.0)
    hq = jnp.concatenate(
        [jnp.maximum(h1[:, 0:896], h1[:, 896:1792]),
         jnp.maximum(h1[:, 1792:2688], h1[:, 2688:3584])], axis=1)
    zg = jnp.zeros((ta, 64), jnp.float32)
    blocks = []
    for u in range(14):
        blocks.append(zg)
        blocks.append(jnp.concatenate(
            [jnp.maximum(hq[:, s2 * 896 + (2 * u) * 32:s2 * 896 + (2 * u) * 32 + 32],
                         hq[:, s2 * 896 + (2 * u + 1) * 32:s2 * 896 + (2 * u + 1) * 32 + 32])
             for s2 in range(2)], axis=1))
        blocks.append(zg)
    bw = 9 * ta
    el = 14 * bw
    zb = jnp.zeros((10 * ta, 64), jnp.float32)
    ppad = jnp.concatenate([zb] + blocks + [zb], axis=0)
    s0 = 10 * ta
    slices = []
    for dx in (-1, 0, 1):
        b0 = s0 + dx * bw
        slices += [ppad[b0 - ta:b0 - ta + el, 32:64],
                   ppad[b0:b0 + el],
                   ppad[b0 + ta:b0 + ta + el, 0:32]]
    p2 = jnp.concatenate(slices, axis=1)
    h2 = jnp.dot(p2, w2_ref[...], preferred_element_type=jnp.float32)
    h2 = jnp.maximum(h2 + b2_ref[...], 0.0)
    hyp = jnp.maximum(h2[:, :64], h2[:, 64:])
    h2p = jnp.concatenate(
        [jnp.maximum(hyp[(2 * u) * bw + ta:(2 * u) * bw + 8 * ta],
                     hyp[(2 * u + 1) * bw + ta:(2 * u + 1) * bw + 8 * ta])
         for u in range(7)] + [jnp.zeros((7 * ta, 64), jnp.float32)],
        axis=1)
    out_ref[...] = h2p.reshape(7, ta, 512)


def _fc_vq_kernel(f_ref, fcw_ref, fcb_ref, cct_ref, fc_out_ref, prob_ref, w_ref):
    t = f_ref.shape[1]
    f = fcb_ref[...]
    for y2p in range(7):
        f = f + jnp.dot(f_ref[y2p], fcw_ref[y2p],
                        preferred_element_type=jnp.float32)
    fc_out_ref[...] = f
    cct = cct_ref[...]
    a2 = jnp.sum(f * f, axis=1, keepdims=True)
    b2 = jnp.sum(cct * cct, axis=0, keepdims=True)
    fg = jnp.dot(f, cct, preferred_element_type=jnp.float32)
    d2 = jnp.maximum(a2 + b2 - 2.0 * fg, 0.0)
    dist = jnp.sqrt(d2 + 1e-12)
    dmin = jnp.min(dist, axis=1, keepdims=True)
    iota = jax.lax.broadcasted_iota(jnp.int32, (t, K), 1)
    label = jnp.min(jnp.where(dist == dmin, iota, K), axis=1, keepdims=True)
    w_ref[...] = (iota == label).astype(jnp.float32)
    z = -dist
    z = z - jnp.max(z, axis=1, keepdims=True)
    e = jnp.exp(z)
    prob_ref[...] = e / jnp.sum(e, axis=1, keepdims=True)


def kernel(x, conv1_w, conv1_b, conv2_w, conv2_b, fc_w, fc_b, cluster_centers):
    b = x.shape[0]
    x5 = x.reshape(b, 7, 4, 28).transpose(1, 0, 2, 3).reshape(7, b, 112)
    w1t = jnp.zeros((6, 28, 2, 2, 28, 32), jnp.float32)
    xs = np.arange(28)
    for yy in range(6):
        for s2 in range(2):
            for s in range(2):
                ky = yy - 2 * s2 - s
                if not 0 <= ky < 3:
                    continue
                for kx in range(3):
                    xv = xs - (kx - 1)
                    valid = (xv >= 0) & (xv < 28)
                    w1t = w1t.at[yy, xs[valid], s2, s, xv[valid], :].set(
                        conv1_w[:, 0, ky, kx])
    w1t = w1t.reshape(168, 3584)
    b1t = jnp.tile(conv1_b, 112).reshape(1, 3584)
    w2r = jnp.zeros((3, 4, 32, 2, 64), jnp.float32)
    for kx in range(3):
        for yy in range(4):
            for s2o in range(2):
                ky = yy - s2o
                if 0 <= ky < 3:
                    w2r = w2r.at[kx, yy, :, s2o, :].set(conv2_w[:, :, ky, kx].T)
    w2r = w2r.reshape(384, 128)
    b2 = jnp.tile(conv2_b, 2).reshape(1, 128)
    fcw = jnp.zeros((7, 512, K), jnp.float32)
    fcw = fcw.at[:, :448].set(
        fc_w.reshape(K, 64, 7, 7).transpose(2, 3, 1, 0).reshape(7, 448, K))
    fcb = fc_b.reshape(1, K)
    cct = cluster_centers.T

    feats_alt = pl.pallas_call(
        _conv_feats_kernel,
        grid=(b // TA,),
        in_specs=[
            pl.BlockSpec((7, TA, 112), lambda i: (0, i, 0)),
            pl.BlockSpec((168, 3584), lambda i: (0, 0)),
            pl.BlockSpec((1, 3584), lambda i: (0, 0)),
            pl.BlockSpec((384, 128), lambda i: (0, 0)),
            pl.BlockSpec((1, 128), lambda i: (0, 0)),
        ],
        out_specs=pl.BlockSpec((7, TA, 512), lambda i: (0, i, 0)),
        out_shape=jax.ShapeDtypeStruct((7, b, 512), jnp.float32),
    )(x5, w1t, b1t, w2r, b2)

    fc_out, prob, w = pl.pallas_call(
        _fc_vq_kernel,
        grid=(b // TB,),
        in_specs=[
            pl.BlockSpec((7, TB, 512), lambda i: (0, i, 0)),
            pl.BlockSpec((7, 512, K), lambda i: (0, 0, 0)),
            pl.BlockSpec((1, K), lambda i: (0, 0)),
            pl.BlockSpec((K, K), lambda i: (0, 0)),
        ],
        out_specs=[
            pl.BlockSpec((TB, K), lambda i: (i, 0)),
            pl.BlockSpec((TB, K), lambda i: (i, 0)),
            pl.BlockSpec((TB, K), lambda i: (i, 0)),
        ],
        out_shape=[
            jax.ShapeDtypeStruct((b, K), jnp.float32),
            jax.ShapeDtypeStruct((b, K), jnp.float32),
            jax.ShapeDtypeStruct((b, K), jnp.float32),
        ],
    )(feats_alt, fcw, fcb, cct)
    return (fc_out, prob, w)


# speedup vs baseline: 1.4186x; 1.0031x over previous
import numpy as np

import jax
import jax.numpy as jnp
from jax.experimental import pallas as pl

TA = 64
TB = 512
K = 512


def _conv_feats_kernel(x_ref, w1t_ref, b1t_ref, w2_ref, b2_ref, out_ref):
    ta = TA
    x = x_ref[...].reshape(7 * ta, 112)
    z1 = jnp.zeros((ta, 112), jnp.float32)
    xe = jnp.concatenate([z1, x, z1], axis=0)
    prev = xe[0:7 * ta, 84:112]
    cur = xe[ta:8 * ta]
    nxt = xe[2 * ta:9 * ta, 0:28]
    p1 = jnp.concatenate([prev, cur, nxt], axis=1)
    h1 = jnp.dot(p1, w1t_ref[...], preferred_element_type=jnp.float32)
    hq = jnp.maximum(jnp.concatenate(
        [jnp.maximum(h1[:, 0:896], h1[:, 896:1792]),
         jnp.maximum(h1[:, 1792:2688], h1[:, 2688:3584])], axis=1)
        + b1t_ref[...], 0.0)
    zg = jnp.zeros((ta, 64), jnp.float32)
    blocks = []
    for u in range(14):
        blocks.append(zg)
        blocks.append(jnp.concatenate(
            [jnp.maximum(hq[:, s2 * 896 + (2 * u) * 32:s2 * 896 + (2 * u) * 32 + 32],
                         hq[:, s2 * 896 + (2 * u + 1) * 32:s2 * 896 + (2 * u + 1) * 32 + 32])
             for s2 in range(2)], axis=1))
        blocks.append(zg)
    bw = 9 * ta
    el = 14 * bw
    zb = jnp.zeros((10 * ta, 64), jnp.float32)
    ppad = jnp.concatenate([zb] + blocks + [zb], axis=0)
    s0 = 10 * ta
    slices = []
    for dx in (-1, 0, 1):
        b0 = s0 + dx * bw
        slices += [ppad[b0 - ta:b0 - ta + el, 32:64],
                   ppad[b0:b0 + el],
                   ppad[b0 + ta:b0 + ta + el, 0:32]]
    p2 = jnp.concatenate(slices, axis=1)
    h2 = jnp.dot(p2, w2_ref[...], preferred_element_type=jnp.float32)
    hyp = jnp.maximum(jnp.maximum(h2[:, :64], h2[:, 64:]) + b2_ref[...], 0.0)
    h2p = jnp.concatenate(
        [jnp.maximum(hyp[(2 * u) * bw + ta:(2 * u) * bw + 8 * ta],
                     hyp[(2 * u + 1) * bw + ta:(2 * u + 1) * bw + 8 * ta])
         for u in range(7)] + [jnp.zeros((7 * ta, 64), jnp.float32)],
        axis=1)
    out_ref[...] = h2p.reshape(7, ta, 512)


def _fc_vq_kernel(f_ref, fcw_ref, fcb_ref, cct_ref, fc_out_ref, prob_ref, w_ref):
    t = f_ref.shape[1]
    f = fcb_ref[...]
    for y2p in range(7):
        f = f + jnp.dot(f_ref[y2p], fcw_ref[y2p],
                        preferred_element_type=jnp.float32)
    fc_out_ref[...] = f
    cct = cct_ref[...]
    a2 = jnp.sum(f * f, axis=1, keepdims=True)
    b2 = jnp.sum(cct * cct, axis=0, keepdims=True)
    fg = jnp.dot(f, cct, preferred_element_type=jnp.float32)
    d2 = jnp.maximum(a2 + b2 - 2.0 * fg, 0.0)
    dist = jnp.sqrt(d2 + 1e-12)
    dmin = jnp.min(dist, axis=1, keepdims=True)
    iota = jax.lax.broadcasted_iota(jnp.int32, (t, K), 1)
    label = jnp.min(jnp.where(dist == dmin, iota, K), axis=1, keepdims=True)
    w_ref[...] = (iota == label).astype(jnp.float32)
    z = -dist
    z = z - jnp.max(z, axis=1, keepdims=True)
    e = jnp.exp(z)
    prob_ref[...] = e / jnp.sum(e, axis=1, keepdims=True)


def kernel(x, conv1_w, conv1_b, conv2_w, conv2_b, fc_w, fc_b, cluster_centers):
    b = x.shape[0]
    x5 = x.reshape(b, 7, 4, 28).transpose(1, 0, 2, 3).reshape(7, b, 112)
    w1t = jnp.zeros((6, 28, 2, 2, 28, 32), jnp.float32)
    xs = np.arange(28)
    for yy in range(6):
        for s2 in range(2):
            for s in range(2):
                ky = yy - 2 * s2 - s
                if not 0 <= ky < 3:
                    continue
                for kx in range(3):
                    xv = xs - (kx - 1)
                    valid = (xv >= 0) & (xv < 28)
                    w1t = w1t.at[yy, xs[valid], s2, s, xv[valid], :].set(
                        conv1_w[:, 0, ky, kx])
    w1t = w1t.reshape(168, 3584)
    b1t = jnp.tile(conv1_b, 56).reshape(1, 1792)
    w2r = jnp.zeros((3, 4, 32, 2, 64), jnp.float32)
    for kx in range(3):
        for yy in range(4):
            for s2o in range(2):
                ky = yy - s2o
                if 0 <= ky < 3:
                    w2r = w2r.at[kx, yy, :, s2o, :].set(conv2_w[:, :, ky, kx].T)
    w2r = w2r.reshape(384, 128)
    b2 = conv2_b.reshape(1, 64)
    fcw = jnp.zeros((7, 512, K), jnp.float32)
    fcw = fcw.at[:, :448].set(
        fc_w.reshape(K, 64, 7, 7).transpose(2, 3, 1, 0).reshape(7, 448, K))
    fcb = fc_b.reshape(1, K)
    cct = cluster_centers.T

    feats_alt = pl.pallas_call(
        _conv_feats_kernel,
        grid=(b // TA,),
        in_specs=[
            pl.BlockSpec((7, TA, 112), lambda i: (0, i, 0)),
            pl.BlockSpec((168, 3584), lambda i: (0, 0)),
            pl.BlockSpec((1, 1792), lambda i: (0, 0)),
            pl.BlockSpec((384, 128), lambda i: (0, 0)),
            pl.BlockSpec((1, 64), lambda i: (0, 0)),
        ],
        out_specs=pl.BlockSpec((7, TA, 512), lambda i: (0, i, 0)),
        out_shape=jax.ShapeDtypeStruct((7, b, 512), jnp.float32),
    )(x5, w1t, b1t, w2r, b2)

    fc_out, prob, w = pl.pallas_call(
        _fc_vq_kernel,
        grid=(b // TB,),
        in_specs=[
            pl.BlockSpec((7, TB, 512), lambda i: (0, i, 0)),
            pl.BlockSpec((7, 512, K), lambda i: (0, 0, 0)),
            pl.BlockSpec((1, K), lambda i: (0, 0)),
            pl.BlockSpec((K, K), lambda i: (0, 0)),
        ],
        out_specs=[
            pl.BlockSpec((TB, K), lambda i: (i, 0)),
            pl.BlockSpec((TB, K), lambda i: (i, 0)),
            pl.BlockSpec((TB, K), lambda i: (i, 0)),
        ],
        out_shape=[
            jax.ShapeDtypeStruct((b, K), jnp.float32),
            jax.ShapeDtypeStruct((b, K), jnp.float32),
            jax.ShapeDtypeStruct((b, K), jnp.float32),
        ],
    )(feats_alt, fcw, fcb, cct)
    return (fc_out, prob, w)
